# baseline (speedup 1.0000x reference)
import functools

import jax
import jax.numpy as jnp
from jax import lax
from jax.experimental import pallas as pl
from jax.experimental.pallas import tpu as pltpu
from jax.experimental.pallas import tpu_sc as plsc

SEQ = 2048
D_MODEL = 1024
D_FF = 4096
N_EXPERTS = 8

BM = 256
NB = SEQ // BM
FBLK = 512
NFF = D_FF // FBLK
T_STEPS = NB + N_EXPERTS - 1
CHUNK = 256
NCHUNK = SEQ // CHUNK


def _router_body(lg_ref, pos_ref, cnt_ref):
    logits = lg_ref[...]
    rowmax = jnp.max(logits, axis=1, keepdims=True)
    e_iota = lax.broadcasted_iota(jnp.int32, (SEQ, N_EXPERTS), 1)
    is_max = logits == rowmax
    e_id = jnp.min(jnp.where(is_max, e_iota, N_EXPERTS), axis=1, keepdims=True)
    onehot = (e_iota == e_id).astype(jnp.float32)

    li = lax.broadcasted_iota(jnp.int32, (CHUNK, CHUNK), 0)
    lj = lax.broadcasted_iota(jnp.int32, (CHUNK, CHUNK), 1)
    ltri = (li >= lj).astype(jnp.float32)
    carry = jnp.zeros((1, N_EXPERTS), jnp.float32)
    pieces = []
    for c in range(NCHUNK):
        oh_c = onehot[c * CHUNK:(c + 1) * CHUNK, :]
        cc = jnp.dot(ltri, oh_c, preferred_element_type=jnp.float32)
        pieces.append(cc + carry)
        carry = carry + cc[CHUNK - 1:CHUNK, :]
    cincl = jnp.concatenate(pieces, axis=0)
    totals = carry

    mi = lax.broadcasted_iota(jnp.int32, (N_EXPERTS, N_EXPERTS), 0)
    mj = lax.broadcasted_iota(jnp.int32, (N_EXPERTS, N_EXPERTS), 1)
    tot_rows = jnp.broadcast_to(totals.reshape(N_EXPERTS, 1),
                                (N_EXPERTS, N_EXPERTS))
    offs = jnp.sum(jnp.where(mi < mj, tot_rows, 0.0), axis=0,
                   keepdims=True)

    pos = jnp.sum(onehot * (cincl + offs), axis=1, keepdims=True) - 1.0
    pos_ref[...] = pos.astype(jnp.int32)
    cnt_ref[...] = totals.astype(jnp.int32)


def _router_call(logits):
    return pl.pallas_call(
        _router_body,
        out_shape=(
            jax.ShapeDtypeStruct((SEQ, 1), jnp.int32),
            jax.ShapeDtypeStruct((1, N_EXPERTS), jnp.int32),
        ),
    )(logits)


_NC = 2
_NS = 16
_NW = _NC * _NS
_RPW = SEQ // _NW

@functools.lru_cache(maxsize=None)
def _sc_kernels():
    mesh = plsc.VectorSubcoreMesh(core_axis_name="c", subcore_axis_name="s")
    scratch = [
        pltpu.VMEM((_RPW,), jnp.int32),
        pltpu.VMEM((_RPW, D_MODEL), jnp.float32),
        pltpu.SemaphoreType.DMA,
    ]
    out_t = jax.ShapeDtypeStruct((SEQ, D_MODEL), jnp.float32)

    @functools.partial(pl.kernel, mesh=mesh, out_type=out_t,
                       scratch_types=scratch)
    def dispatch(x_hbm, pos_hbm, xs_hbm, idx_v, rows_v, sem):
        wid = lax.axis_index("s") * _NC + lax.axis_index("c")
        base = wid * _RPW
        pltpu.sync_copy(pos_hbm.at[pl.ds(base, _RPW)], idx_v)
        pltpu.sync_copy(x_hbm.at[pl.ds(base, _RPW)], rows_v)
        pltpu.async_copy(rows_v, xs_hbm.at[idx_v], sem).wait()

    @functools.partial(pl.kernel, mesh=mesh, out_type=out_t,
                       scratch_types=scratch)
    def combine(ys_hbm, pos_hbm, out_hbm, idx_v, rows_v, sem):
        wid = lax.axis_index("s") * _NC + lax.axis_index("c")
        base = wid * _RPW
        pltpu.sync_copy(pos_hbm.at[pl.ds(base, _RPW)], idx_v)
        pltpu.async_copy(ys_hbm.at[idx_v], rows_v, sem).wait()
        pltpu.sync_copy(rows_v, out_hbm.at[pl.ds(base, _RPW)])

    return dispatch, combine


def _mm_body(blk_ref, exp_ref, offs_ref, val_ref,
             xs_ref, g_ref, u_ref, d_ref, out_ref):
    f = pl.program_id(0)
    t = pl.program_id(1)
    e = exp_ref[t]
    blk = blk_ref[t]
    lo = offs_ref[e]
    hi = offs_ref[e + 1]
    row = blk * BM + lax.broadcasted_iota(jnp.int32, (BM, 1), 0)
    active = (row >= lo) & (row < hi) & (val_ref[t] > 0)
    x = jnp.where(active, xs_ref[pl.ds(blk * BM, BM), :], 0.0)
    x = x.astype(g_ref.dtype)
    g = jnp.dot(x, g_ref[0], preferred_element_type=jnp.float32)
    u = jnp.dot(x, u_ref[0], preferred_element_type=jnp.float32)
    h = (g * jax.nn.sigmoid(g)) * u
    y = jnp.dot(h.astype(d_ref.dtype), d_ref[0],
                preferred_element_type=jnp.float32)

    tm1 = jnp.maximum(t - 1, 0)
    first = (f == 0) & ((t == 0) | (blk != blk_ref[tm1]))

    @pl.when(first)
    def _():
        out_ref[pl.ds(blk * BM, BM), :] = y

    @pl.when(jnp.logical_not(first))
    def _():
        out_ref[pl.ds(blk * BM, BM), :] += y


def _mm_call(blk, exp, offs, valid, xs, gate_w, up_w, down_w):
    grid_spec = pltpu.PrefetchScalarGridSpec(
        num_scalar_prefetch=4,
        grid=(NFF, T_STEPS),
        in_specs=[
            pl.BlockSpec((SEQ, D_MODEL), lambda f, t, b, e, o, v: (0, 0)),
            pl.BlockSpec((1, D_MODEL, FBLK), lambda f, t, b, e, o, v: (e[t], 0, f)),
            pl.BlockSpec((1, D_MODEL, FBLK), lambda f, t, b, e, o, v: (e[t], 0, f)),
            pl.BlockSpec((1, FBLK, D_MODEL), lambda f, t, b, e, o, v: (e[t], f, 0)),
        ],
        out_specs=pl.BlockSpec((SEQ, D_MODEL), lambda f, t, b, e, o, v: (0, 0)),
    )
    return pl.pallas_call(
        _mm_body,
        grid_spec=grid_spec,
        out_shape=jax.ShapeDtypeStruct((SEQ, D_MODEL), jnp.float32),
        compiler_params=pltpu.CompilerParams(
            dimension_semantics=("arbitrary", "arbitrary"),
        ),
    )(blk, exp, offs, valid, xs, gate_w, up_w, down_w)


def _schedule(counts):
    counts = counts.reshape(N_EXPERTS).astype(jnp.int32)
    offs = jnp.concatenate(
        [jnp.zeros((1,), jnp.int32), jnp.cumsum(counts, dtype=jnp.int32)])
    first_blk = offs[:N_EXPERTS] // BM
    last_blk = jnp.where(counts > 0, (offs[1:] - 1) // BM, first_blk)
    ntiles = jnp.where(counts > 0, last_blk - first_blk + 1, 0)
    tstart = jnp.concatenate(
        [jnp.zeros((1,), jnp.int32), jnp.cumsum(ntiles, dtype=jnp.int32)])
    t = jnp.arange(T_STEPS, dtype=jnp.int32)
    e = jnp.clip(jnp.searchsorted(tstart, t, side="right").astype(jnp.int32) - 1,
                 0, N_EXPERTS - 1)
    blk = first_blk[e] + (t - tstart[e])
    valid = (t < tstart[N_EXPERTS]).astype(jnp.int32)
    blk = jnp.where(valid > 0, blk, NB - 1).astype(jnp.int32)
    e = jnp.where(valid > 0, e, N_EXPERTS - 1).astype(jnp.int32)
    return blk, e, offs, valid


def kernel(hidden_states, router_w, gate_w, up_w, down_w):
    x = hidden_states.reshape(SEQ, D_MODEL)
    logits = jnp.einsum('sbd,de->sbe', hidden_states, router_w)
    pos2, cnt = _router_call(logits.reshape(SEQ, N_EXPERTS))
    pos = pos2.reshape(SEQ)
    blk, exp, offs, valid = _schedule(cnt)
    dispatch, combine = _sc_kernels()
    xs = dispatch(x, pos)
    ys = _mm_call(blk, exp, offs, valid, xs,
                  gate_w.astype(jnp.bfloat16), up_w.astype(jnp.bfloat16),
                  down_w.astype(jnp.bfloat16))
    out = combine(ys, pos)
    return out.reshape(SEQ, 1, D_MODEL)


# speedup vs baseline: 1.6288x; 1.6288x over previous
import functools

import jax
import jax.numpy as jnp
from jax import lax
from jax.experimental import pallas as pl
from jax.experimental.pallas import tpu as pltpu
from jax.experimental.pallas import tpu_sc as plsc

SEQ = 2048
D_MODEL = 1024
D_FF = 4096
N_EXPERTS = 8

BM = 256
NB = SEQ // BM
FBLK = 1024
NFF = D_FF // FBLK
T_STEPS = NB + N_EXPERTS - 1
CHUNK = 256
NCHUNK = SEQ // CHUNK


def _router_body(lg_ref, pos_ref, cnt_ref):
    logits = lg_ref[...]
    rowmax = jnp.max(logits, axis=1, keepdims=True)
    e_iota = lax.broadcasted_iota(jnp.int32, (SEQ, N_EXPERTS), 1)
    is_max = logits == rowmax
    e_id = jnp.min(jnp.where(is_max, e_iota, N_EXPERTS), axis=1, keepdims=True)
    onehot = (e_iota == e_id).astype(jnp.float32)

    li = lax.broadcasted_iota(jnp.int32, (CHUNK, CHUNK), 0)
    lj = lax.broadcasted_iota(jnp.int32, (CHUNK, CHUNK), 1)
    ltri = (li >= lj).astype(jnp.float32)
    carry = jnp.zeros((1, N_EXPERTS), jnp.float32)
    pieces = []
    for c in range(NCHUNK):
        oh_c = onehot[c * CHUNK:(c + 1) * CHUNK, :]
        cc = jnp.dot(ltri, oh_c, preferred_element_type=jnp.float32)
        pieces.append(cc + carry)
        carry = carry + cc[CHUNK - 1:CHUNK, :]
    cincl = jnp.concatenate(pieces, axis=0)
    totals = carry

    mi = lax.broadcasted_iota(jnp.int32, (N_EXPERTS, N_EXPERTS), 0)
    mj = lax.broadcasted_iota(jnp.int32, (N_EXPERTS, N_EXPERTS), 1)
    tot_rows = jnp.broadcast_to(totals.reshape(N_EXPERTS, 1),
                                (N_EXPERTS, N_EXPERTS))
    offs = jnp.sum(jnp.where(mi < mj, tot_rows, 0.0), axis=0,
                   keepdims=True)

    pos = jnp.sum(onehot * (cincl + offs), axis=1, keepdims=True) - 1.0
    pos_ref[...] = pos.astype(jnp.int32)
    cnt_ref[...] = totals.astype(jnp.int32)


def _router_call(logits):
    return pl.pallas_call(
        _router_body,
        out_shape=(
            jax.ShapeDtypeStruct((SEQ, 1), jnp.int32),
            jax.ShapeDtypeStruct((1, N_EXPERTS), jnp.int32),
        ),
    )(logits)


_NC = 2
_NS = 16
_NW = _NC * _NS
_RPW = SEQ // _NW

@functools.lru_cache(maxsize=None)
def _sc_kernels():
    mesh = plsc.VectorSubcoreMesh(core_axis_name="c", subcore_axis_name="s")
    scratch = [
        pltpu.VMEM((_RPW,), jnp.int32),
        pltpu.VMEM((_RPW, D_MODEL), jnp.float32),
        pltpu.SemaphoreType.DMA,
    ]
    out_t = jax.ShapeDtypeStruct((SEQ, D_MODEL), jnp.float32)

    @functools.partial(pl.kernel, mesh=mesh, out_type=out_t,
                       scratch_types=scratch)
    def dispatch(x_hbm, pos_hbm, xs_hbm, idx_v, rows_v, sem):
        wid = lax.axis_index("s") * _NC + lax.axis_index("c")
        base = wid * _RPW
        pltpu.sync_copy(pos_hbm.at[pl.ds(base, _RPW)], idx_v)
        pltpu.sync_copy(x_hbm.at[pl.ds(base, _RPW)], rows_v)
        pltpu.async_copy(rows_v, xs_hbm.at[idx_v], sem).wait()

    @functools.partial(pl.kernel, mesh=mesh, out_type=out_t,
                       scratch_types=scratch)
    def combine(ys_hbm, pos_hbm, out_hbm, idx_v, rows_v, sem):
        wid = lax.axis_index("s") * _NC + lax.axis_index("c")
        base = wid * _RPW
        pltpu.sync_copy(pos_hbm.at[pl.ds(base, _RPW)], idx_v)
        pltpu.async_copy(ys_hbm.at[idx_v], rows_v, sem).wait()
        pltpu.sync_copy(rows_v, out_hbm.at[pl.ds(base, _RPW)])

    return dispatch, combine


def _mm_body(blk_ref, exp_ref, offs_ref, val_ref,
             xs_ref, g_ref, u_ref, d_ref, out_ref):
    f = pl.program_id(0)
    t = pl.program_id(1)
    e = exp_ref[t]
    blk = blk_ref[t]
    lo = offs_ref[e]
    hi = offs_ref[e + 1]
    row = blk * BM + lax.broadcasted_iota(jnp.int32, (BM, 1), 0)
    active = (row >= lo) & (row < hi) & (val_ref[t] > 0)
    x = jnp.where(active, xs_ref[pl.ds(blk * BM, BM), :], 0.0)
    g = jnp.dot(x, g_ref[0], preferred_element_type=jnp.float32)
    u = jnp.dot(x, u_ref[0], preferred_element_type=jnp.float32)
    h = (g * jax.nn.sigmoid(g)) * u
    y = jnp.dot(h, d_ref[0], preferred_element_type=jnp.float32)

    tm1 = jnp.maximum(t - 1, 0)
    first = (f == 0) & ((t == 0) | (blk != blk_ref[tm1]))

    @pl.when(first)
    def _():
        out_ref[pl.ds(blk * BM, BM), :] = y

    @pl.when(jnp.logical_not(first))
    def _():
        out_ref[pl.ds(blk * BM, BM), :] += y


def _mm_call(blk, exp, offs, valid, xs, gate_w, up_w, down_w):
    grid_spec = pltpu.PrefetchScalarGridSpec(
        num_scalar_prefetch=4,
        grid=(NFF, T_STEPS),
        in_specs=[
            pl.BlockSpec((SEQ, D_MODEL), lambda f, t, b, e, o, v: (0, 0)),
            pl.BlockSpec((1, D_MODEL, FBLK), lambda f, t, b, e, o, v: (e[t], 0, f)),
            pl.BlockSpec((1, D_MODEL, FBLK), lambda f, t, b, e, o, v: (e[t], 0, f)),
            pl.BlockSpec((1, FBLK, D_MODEL), lambda f, t, b, e, o, v: (e[t], f, 0)),
        ],
        out_specs=pl.BlockSpec((SEQ, D_MODEL), lambda f, t, b, e, o, v: (0, 0)),
    )
    return pl.pallas_call(
        _mm_body,
        grid_spec=grid_spec,
        out_shape=jax.ShapeDtypeStruct((SEQ, D_MODEL), jnp.float32),
        compiler_params=pltpu.CompilerParams(
            dimension_semantics=("arbitrary", "arbitrary"),
        ),
    )(blk, exp, offs, valid, xs, gate_w, up_w, down_w)


def _schedule(counts):
    counts = counts.reshape(N_EXPERTS).astype(jnp.int32)
    offs = jnp.concatenate(
        [jnp.zeros((1,), jnp.int32), jnp.cumsum(counts, dtype=jnp.int32)])
    first_blk = offs[:N_EXPERTS] // BM
    last_blk = jnp.where(counts > 0, (offs[1:] - 1) // BM, first_blk)
    ntiles = jnp.where(counts > 0, last_blk - first_blk + 1, 0)
    tstart = jnp.concatenate(
        [jnp.zeros((1,), jnp.int32), jnp.cumsum(ntiles, dtype=jnp.int32)])
    t = jnp.arange(T_STEPS, dtype=jnp.int32)
    e = jnp.clip(jnp.searchsorted(tstart, t, side="right").astype(jnp.int32) - 1,
                 0, N_EXPERTS - 1)
    blk = first_blk[e] + (t - tstart[e])
    valid = (t < tstart[N_EXPERTS]).astype(jnp.int32)
    blk = jnp.where(valid > 0, blk, NB - 1).astype(jnp.int32)
    e = jnp.where(valid > 0, e, N_EXPERTS - 1).astype(jnp.int32)
    return blk, e, offs, valid


def kernel(hidden_states, router_w, gate_w, up_w, down_w):
    x = hidden_states.reshape(SEQ, D_MODEL)
    logits = jnp.einsum('sbd,de->sbe', hidden_states, router_w)
    pos2, cnt = _router_call(logits.reshape(SEQ, N_EXPERTS))
    pos = pos2.reshape(SEQ)
    blk, exp, offs, valid = _schedule(cnt)
    dispatch, combine = _sc_kernels()
    xs = dispatch(x, pos)
    ys = _mm_call(blk, exp, offs, valid, xs, gate_w, up_w, down_w)
    out = combine(ys, pos)
    return out.reshape(SEQ, 1, D_MODEL)
